# baseline (speedup 1.0000x reference)
import functools

import jax
import jax.numpy as jnp
from jax import lax
from jax.experimental import pallas as pl
from jax.experimental.pallas import tpu as pltpu
from jax.experimental.pallas import tpu_sc as plsc

N = 10000
NP = 10240
D = 128
E = 320000
NC = 2
NS = 16
NW = NC * NS
CH = 125
NCHT = E // CH
NCH = NCHT // NW
CHH = 40
RPT = NP // NS
BLK = 1024
GRID = NP // BLK


DCH = 80
EPW = E // NW
DNCH = EPW // DCH


def _deg_body(dst_hbm, out_hbm, dbuf, buf, degsh, dsem):
    c = lax.axis_index("c")
    s = lax.axis_index("s")
    wid = c * NS + s
    row0 = s * RPT

    zero16 = jnp.zeros((16,), jnp.float32)
    one16 = jnp.ones((16,), jnp.float32)

    def fill(i, _):
        buf[pl.ds(i * 16, 16)] = zero16
        buf[pl.ds(128 + i * 16, 16)] = one16
        return 0

    lax.fori_loop(0, 128 // 16, fill, 0)
    for r in range(RPT // 128):
        pltpu.sync_copy(buf.at[pl.ds(0, 128)],
                        degsh.at[pl.ds(row0 + r * 128, 128)])
    pltpu.sync_copy(dst_hbm.at[pl.ds(wid * EPW, EPW)], dbuf)
    plsc.subcore_barrier()

    K = 5

    def body(g, _):
        for k in range(K):
            pltpu.async_copy(buf.at[pl.ds(128, DCH)],
                             degsh.at[dbuf.at[pl.ds((g * K + k) * DCH, DCH)]],
                             dsem, add=True)
        for k in range(K):
            pltpu.make_async_copy(
                buf.at[pl.ds(128, DCH)],
                degsh.at[dbuf.at[pl.ds((g * K + k) * DCH, DCH)]], dsem).wait()
        return 0

    lax.fori_loop(0, DNCH // K, body, 0)
    plsc.subcore_barrier()
    pltpu.sync_copy(degsh.at[pl.ds(row0, RPT)], out_hbm.at[c, pl.ds(row0, RPT)])


_deg_kernel = functools.partial(
    pl.kernel,
    out_type=jax.ShapeDtypeStruct((NC, NP), jnp.float32),
    mesh=plsc.VectorSubcoreMesh(core_axis_name="c", subcore_axis_name="s"),
    scratch_types=[
        pltpu.VMEM((EPW,), jnp.int32),
        pltpu.VMEM((256,), jnp.float32),
        pltpu.VMEM_SHARED((NP,), jnp.float32),
        pltpu.SemaphoreType.DMA,
    ],
)(_deg_body)


def _scatter_body(hs_hbm, ei_hbm, out_hbm,
                  sbuf, dbuf, rows0, rows1, aggsh, sem0, sem1, ssem0, ssem1):
    c = lax.axis_index("c")
    s = lax.axis_index("s")
    wid = c * NS + s
    row0 = s * RPT

    zero16 = jnp.zeros((16,), jnp.float32)

    def zfill(i, _):
        for k in range(D // 16):
            rows0[i, pl.ds(k * 16, 16)] = zero16
        return 0

    lax.fori_loop(0, 80, zfill, 0)
    for r in range(RPT // 80):
        pltpu.sync_copy(rows0.at[pl.ds(0, 80)],
                        aggsh.at[pl.ds(row0 + r * 80, 80)])

    plsc.subcore_barrier()

    for h in range(NCH // CHH):
        base = wid * NCH + h * CHH
        pltpu.sync_copy(ei_hbm.at[0, pl.ds(base, CHH)], sbuf)
        pltpu.sync_copy(ei_hbm.at[1, pl.ds(base, CHH)], dbuf)
        pltpu.async_copy(hs_hbm.at[sbuf.at[0]], rows0, sem0)

        def body(t, _):
            j0 = 2 * t
            pltpu.async_copy(hs_hbm.at[sbuf.at[j0 + 1]], rows1, sem1)
            pltpu.make_async_copy(hs_hbm.at[sbuf.at[j0]], rows0, sem0).wait()
            pltpu.sync_copy(rows0, aggsh.at[dbuf.at[j0]], add=True)

            @pl.when(t + 1 < CHH // 2)
            def _():
                pltpu.async_copy(hs_hbm.at[sbuf.at[j0 + 2]], rows0, sem0)

            pltpu.make_async_copy(hs_hbm.at[sbuf.at[j0 + 1]], rows1, sem1).wait()
            pltpu.sync_copy(rows1, aggsh.at[dbuf.at[j0 + 1]], add=True)
            return 0

        lax.fori_loop(0, CHH // 2, body, 0)
    plsc.subcore_barrier()
    pltpu.sync_copy(aggsh.at[pl.ds(row0, RPT)], out_hbm.at[c, pl.ds(row0, RPT)])


_scatter_kernel = functools.partial(
    pl.kernel,
    out_type=jax.ShapeDtypeStruct((NC, NP, D), jnp.float32),
    mesh=plsc.VectorSubcoreMesh(core_axis_name="c", subcore_axis_name="s"),
    scratch_types=[
        pltpu.VMEM((CHH, CH), jnp.int32),
        pltpu.VMEM((CHH, CH), jnp.int32),
        pltpu.VMEM((CH, D), jnp.float32),
        pltpu.VMEM((CH, D), jnp.float32),
        pltpu.VMEM_SHARED((NP, D), jnp.float32),
        pltpu.SemaphoreType.DMA,
        pltpu.SemaphoreType.DMA,
        pltpu.SemaphoreType.DMA,
        pltpu.SemaphoreType.DMA,
    ],
)(_scatter_body)


def _dis(degp_ref):
    deg = degp_ref[0] + degp_ref[1] + 1.0
    return lax.rsqrt(deg).reshape(BLK, 1)


def _tc1_body(x_ref, w_ref, degp_ref, hs_ref):
    h = jnp.dot(x_ref[...], w_ref[...], preferred_element_type=jnp.float32)
    hs_ref[...] = h * _dis(degp_ref)


def _tc1(x, w1, degp):
    return pl.pallas_call(
        _tc1_body,
        grid=(GRID,),
        in_specs=[
            pl.BlockSpec((BLK, D), lambda i: (i, 0)),
            pl.BlockSpec((D, D), lambda i: (0, 0)),
            pl.BlockSpec((NC, BLK), lambda i: (0, i)),
        ],
        out_specs=pl.BlockSpec((BLK, D), lambda i: (i, 0)),
        out_shape=jax.ShapeDtypeStruct((NP, D), jnp.float32),
    )(x, w1, degp)


def _tc2_body(agg_ref, hs_ref_in, degp_ref, b_ref, w_ref, hs_ref):
    dis = _dis(degp_ref)
    a = agg_ref[0] + agg_ref[1] + hs_ref_in[...]
    z = jnp.maximum(a * dis + b_ref[...], 0.0)
    h = jnp.dot(z, w_ref[...], preferred_element_type=jnp.float32)
    hs_ref[...] = h * dis


def _tc2(agg, hs, degp, b1, w2):
    return pl.pallas_call(
        _tc2_body,
        grid=(GRID,),
        in_specs=[
            pl.BlockSpec((NC, BLK, D), lambda i: (0, i, 0)),
            pl.BlockSpec((BLK, D), lambda i: (i, 0)),
            pl.BlockSpec((NC, BLK), lambda i: (0, i)),
            pl.BlockSpec((1, D), lambda i: (0, 0)),
            pl.BlockSpec((D, D), lambda i: (0, 0)),
        ],
        out_specs=pl.BlockSpec((BLK, D), lambda i: (i, 0)),
        out_shape=jax.ShapeDtypeStruct((NP, D), jnp.float32),
    )(agg, hs, degp, b1, w2)


def _tc3_body(agg_ref, hs_ref_in, degp_ref, b_ref, out_ref):
    a = agg_ref[0] + agg_ref[1] + hs_ref_in[...]
    out_ref[...] = jnp.maximum(a * _dis(degp_ref) + b_ref[...], 0.0)


def _tc3(agg, hs, degp, b2):
    return pl.pallas_call(
        _tc3_body,
        grid=(GRID,),
        in_specs=[
            pl.BlockSpec((NC, BLK, D), lambda i: (0, i, 0)),
            pl.BlockSpec((BLK, D), lambda i: (i, 0)),
            pl.BlockSpec((NC, BLK), lambda i: (0, i)),
            pl.BlockSpec((1, D), lambda i: (0, 0)),
        ],
        out_specs=pl.BlockSpec((BLK, D), lambda i: (i, 0)),
        out_shape=jax.ShapeDtypeStruct((N, D), jnp.float32),
    )(agg, hs, degp, b2)


def kernel(x, edge_index, W1, b1, W2, b2):
    ei2 = edge_index.astype(jnp.int32)
    ei = ei2.reshape(2, NCHT, CH)
    b1r = b1.reshape(1, D)
    b2r = b2.reshape(1, D)

    degp = _deg_kernel(ei2[1])
    hs1 = _tc1(x, W1, degp)
    agg1 = _scatter_kernel(hs1, ei)
    hs2 = _tc2(agg1, hs1, degp, b1r, W2)
    agg2 = _scatter_kernel(hs2, ei)
    return _tc3(agg2, hs2, degp, b2r)


# speedup vs baseline: 1.0194x; 1.0194x over previous
import functools

import jax
import jax.numpy as jnp
from jax import lax
from jax.experimental import pallas as pl
from jax.experimental.pallas import tpu as pltpu
from jax.experimental.pallas import tpu_sc as plsc

N = 10000
NP = 10240
D = 128
E = 320000
NC = 2
NS = 16
NW = NC * NS
CH = 125
NCHT = E // CH
NCH = NCHT // NW
CHH = 40
RPT = NP // NS
BLK = 1024
GRID = NP // BLK


def _deg_body(ei_hbm, out_hbm, dbuf, buf, degsh, dsem):
    c = lax.axis_index("c")
    s = lax.axis_index("s")
    wid = c * NS + s
    row0 = s * RPT

    zero16 = jnp.zeros((16,), jnp.float32)
    one16 = jnp.ones((16,), jnp.float32)

    def fill(i, _):
        buf[pl.ds(i * 16, 16)] = zero16
        buf[pl.ds(128 + i * 16, 16)] = one16
        return 0

    lax.fori_loop(0, 128 // 16, fill, 0)
    for r in range(RPT // 128):
        pltpu.sync_copy(buf.at[pl.ds(0, 128)],
                        degsh.at[pl.ds(row0 + r * 128, 128)])
    pltpu.sync_copy(ei_hbm.at[1, pl.ds(wid * NCH, NCH)], dbuf)
    plsc.subcore_barrier()

    K = 8

    def body(g, _):
        for k in range(K):
            pltpu.async_copy(buf.at[pl.ds(128, CH)],
                             degsh.at[dbuf.at[g * K + k]], dsem, add=True)
        for k in range(K):
            pltpu.make_async_copy(buf.at[pl.ds(128, CH)],
                                  degsh.at[dbuf.at[g * K + k]], dsem).wait()
        return 0

    lax.fori_loop(0, NCH // K, body, 0)
    plsc.subcore_barrier()
    pltpu.sync_copy(degsh.at[pl.ds(row0, RPT)], out_hbm.at[c, pl.ds(row0, RPT)])


_deg_kernel = functools.partial(
    pl.kernel,
    out_type=jax.ShapeDtypeStruct((NC, NP), jnp.float32),
    mesh=plsc.VectorSubcoreMesh(core_axis_name="c", subcore_axis_name="s"),
    scratch_types=[
        pltpu.VMEM((NCH, CH), jnp.int32),
        pltpu.VMEM((256,), jnp.float32),
        pltpu.VMEM_SHARED((NP,), jnp.float32),
        pltpu.SemaphoreType.DMA,
    ],
)(_deg_body)


def _scatter_body(hs_hbm, ei_hbm, out_hbm,
                  sbuf, dbuf, rows0, rows1, aggsh, sem0, sem1, ssem0, ssem1):
    c = lax.axis_index("c")
    s = lax.axis_index("s")
    wid = c * NS + s
    row0 = s * RPT

    zero16 = jnp.zeros((16,), jnp.float32)

    def zfill(i, _):
        for k in range(D // 16):
            rows0[i, pl.ds(k * 16, 16)] = zero16
        return 0

    lax.fori_loop(0, 80, zfill, 0)
    for r in range(RPT // 80):
        pltpu.sync_copy(rows0.at[pl.ds(0, 80)],
                        aggsh.at[pl.ds(row0 + r * 80, 80)])

    plsc.subcore_barrier()

    for h in range(NCH // CHH):
        base = wid * NCH + h * CHH
        pltpu.sync_copy(ei_hbm.at[0, pl.ds(base, CHH)], sbuf)
        pltpu.sync_copy(ei_hbm.at[1, pl.ds(base, CHH)], dbuf)
        pltpu.async_copy(hs_hbm.at[sbuf.at[0]], rows0, sem0)

        def body(t, _):
            j0 = 2 * t
            pltpu.async_copy(hs_hbm.at[sbuf.at[j0 + 1]], rows1, sem1)
            pltpu.make_async_copy(hs_hbm.at[sbuf.at[j0]], rows0, sem0).wait()
            pltpu.sync_copy(rows0, aggsh.at[dbuf.at[j0]], add=True)

            @pl.when(t + 1 < CHH // 2)
            def _():
                pltpu.async_copy(hs_hbm.at[sbuf.at[j0 + 2]], rows0, sem0)

            pltpu.make_async_copy(hs_hbm.at[sbuf.at[j0 + 1]], rows1, sem1).wait()
            pltpu.sync_copy(rows1, aggsh.at[dbuf.at[j0 + 1]], add=True)
            return 0

        lax.fori_loop(0, CHH // 2, body, 0)
    plsc.subcore_barrier()
    pltpu.sync_copy(aggsh.at[pl.ds(row0, RPT)], out_hbm.at[c, pl.ds(row0, RPT)])


_scatter_kernel = functools.partial(
    pl.kernel,
    out_type=jax.ShapeDtypeStruct((NC, NP, D), jnp.float32),
    mesh=plsc.VectorSubcoreMesh(core_axis_name="c", subcore_axis_name="s"),
    scratch_types=[
        pltpu.VMEM((CHH, CH), jnp.int32),
        pltpu.VMEM((CHH, CH), jnp.int32),
        pltpu.VMEM((CH, D), jnp.float32),
        pltpu.VMEM((CH, D), jnp.float32),
        pltpu.VMEM_SHARED((NP, D), jnp.float32),
        pltpu.SemaphoreType.DMA,
        pltpu.SemaphoreType.DMA,
        pltpu.SemaphoreType.DMA,
        pltpu.SemaphoreType.DMA,
    ],
)(_scatter_body)


def _dis(degp_ref):
    deg = degp_ref[0] + degp_ref[1] + 1.0
    return lax.rsqrt(deg).reshape(BLK, 1)


def _tc0_body(x_ref, w_ref, h_ref):
    h_ref[...] = jnp.dot(x_ref[...], w_ref[...],
                         preferred_element_type=jnp.float32)


def _tc0(x, w1):
    return pl.pallas_call(
        _tc0_body,
        grid=(GRID,),
        in_specs=[
            pl.BlockSpec((BLK, D), lambda i: (i, 0)),
            pl.BlockSpec((D, D), lambda i: (0, 0)),
        ],
        out_specs=pl.BlockSpec((BLK, D), lambda i: (i, 0)),
        out_shape=jax.ShapeDtypeStruct((NP, D), jnp.float32),
    )(x, w1)


def _tc1_body(h_ref, degp_ref, hs_ref):
    hs_ref[...] = h_ref[...] * _dis(degp_ref)


def _tc1(h, degp):
    return pl.pallas_call(
        _tc1_body,
        grid=(GRID,),
        in_specs=[
            pl.BlockSpec((BLK, D), lambda i: (i, 0)),
            pl.BlockSpec((NC, BLK), lambda i: (0, i)),
        ],
        out_specs=pl.BlockSpec((BLK, D), lambda i: (i, 0)),
        out_shape=jax.ShapeDtypeStruct((NP, D), jnp.float32),
    )(h, degp)


def _tc2_body(agg_ref, hs_ref_in, degp_ref, b_ref, w_ref, hs_ref):
    dis = _dis(degp_ref)
    a = agg_ref[0] + agg_ref[1] + hs_ref_in[...]
    z = jnp.maximum(a * dis + b_ref[...], 0.0)
    h = jnp.dot(z, w_ref[...], preferred_element_type=jnp.float32)
    hs_ref[...] = h * dis


def _tc2(agg, hs, degp, b1, w2):
    return pl.pallas_call(
        _tc2_body,
        grid=(GRID,),
        in_specs=[
            pl.BlockSpec((NC, BLK, D), lambda i: (0, i, 0)),
            pl.BlockSpec((BLK, D), lambda i: (i, 0)),
            pl.BlockSpec((NC, BLK), lambda i: (0, i)),
            pl.BlockSpec((1, D), lambda i: (0, 0)),
            pl.BlockSpec((D, D), lambda i: (0, 0)),
        ],
        out_specs=pl.BlockSpec((BLK, D), lambda i: (i, 0)),
        out_shape=jax.ShapeDtypeStruct((NP, D), jnp.float32),
    )(agg, hs, degp, b1, w2)


def _tc3_body(agg_ref, hs_ref_in, degp_ref, b_ref, out_ref):
    a = agg_ref[0] + agg_ref[1] + hs_ref_in[...]
    out_ref[...] = jnp.maximum(a * _dis(degp_ref) + b_ref[...], 0.0)


def _tc3(agg, hs, degp, b2):
    return pl.pallas_call(
        _tc3_body,
        grid=(GRID,),
        in_specs=[
            pl.BlockSpec((NC, BLK, D), lambda i: (0, i, 0)),
            pl.BlockSpec((BLK, D), lambda i: (i, 0)),
            pl.BlockSpec((NC, BLK), lambda i: (0, i)),
            pl.BlockSpec((1, D), lambda i: (0, 0)),
        ],
        out_specs=pl.BlockSpec((BLK, D), lambda i: (i, 0)),
        out_shape=jax.ShapeDtypeStruct((N, D), jnp.float32),
    )(agg, hs, degp, b2)


def kernel(x, edge_index, W1, b1, W2, b2):
    ei2 = edge_index.astype(jnp.int32)
    ei = ei2.reshape(2, NCHT, CH)
    b1r = b1.reshape(1, D)
    b2r = b2.reshape(1, D)

    h1 = _tc0(x, W1)
    degp = _deg_kernel(ei)
    hs1 = _tc1(h1, degp)
    agg1 = _scatter_kernel(hs1, ei)
    hs2 = _tc2(agg1, hs1, degp, b1r, W2)
    agg2 = _scatter_kernel(hs2, ei)
    return _tc3(agg2, hs2, degp, b2r)


# speedup vs baseline: 1.0286x; 1.0090x over previous
import functools

import jax
import jax.numpy as jnp
from jax import lax
from jax.experimental import pallas as pl
from jax.experimental.pallas import tpu as pltpu
from jax.experimental.pallas import tpu_sc as plsc

N = 10000
NP = 10240
D = 128
E = 320000
NC = 2
NS = 16
NW = NC * NS
CH = 125
NCHT = E // CH
NCH = NCHT // NW
CHH = 40
RPT = NP // NS
BLK = 1024
GRID = NP // BLK


def _deg_body(ei_hbm, out_hbm, dbuf, buf, degsh, dsem):
    c = lax.axis_index("c")
    s = lax.axis_index("s")
    wid = c * NS + s
    row0 = s * RPT

    zero16 = jnp.zeros((16,), jnp.float32)
    one16 = jnp.ones((16,), jnp.float32)

    def fill(i, _):
        buf[pl.ds(i * 16, 16)] = zero16
        buf[pl.ds(128 + i * 16, 16)] = one16
        return 0

    lax.fori_loop(0, 128 // 16, fill, 0)
    for r in range(RPT // 128):
        pltpu.sync_copy(buf.at[pl.ds(0, 128)],
                        degsh.at[pl.ds(row0 + r * 128, 128)])
    pltpu.sync_copy(ei_hbm.at[1, pl.ds(wid * NCH, NCH)], dbuf)
    plsc.subcore_barrier()

    K = 8

    def body(g, _):
        for k in range(K):
            pltpu.async_copy(buf.at[pl.ds(128, CH)],
                             degsh.at[dbuf.at[g * K + k]], dsem, add=True)
        for k in range(K):
            pltpu.make_async_copy(buf.at[pl.ds(128, CH)],
                                  degsh.at[dbuf.at[g * K + k]], dsem).wait()
        return 0

    lax.fori_loop(0, NCH // K, body, 0)
    plsc.subcore_barrier()
    pltpu.sync_copy(degsh.at[pl.ds(row0, RPT)], out_hbm.at[c, pl.ds(row0, RPT)])


_deg_kernel = functools.partial(
    pl.kernel,
    out_type=jax.ShapeDtypeStruct((NC, NP), jnp.float32),
    mesh=plsc.VectorSubcoreMesh(core_axis_name="c", subcore_axis_name="s"),
    scratch_types=[
        pltpu.VMEM((NCH, CH), jnp.int32),
        pltpu.VMEM((256,), jnp.float32),
        pltpu.VMEM_SHARED((NP,), jnp.float32),
        pltpu.SemaphoreType.DMA,
    ],
)(_deg_body)


def _scatter_body(hs_hbm, ei_hbm, out_hbm,
                  sbuf, dbuf, rows0, rows1, aggsh, sem0, sem1, ssem0, ssem1):
    c = lax.axis_index("c")
    s = lax.axis_index("s")
    wid = c * NS + s
    row0 = s * RPT

    zero16 = jnp.zeros((16,), jnp.float32)

    def zfill(i, _):
        for k in range(D // 16):
            rows0[i, pl.ds(k * 16, 16)] = zero16
        return 0

    lax.fori_loop(0, 80, zfill, 0)
    for r in range(RPT // 80):
        pltpu.sync_copy(rows0.at[pl.ds(0, 80)],
                        aggsh.at[pl.ds(row0 + r * 80, 80)])

    plsc.subcore_barrier()

    for h in range(NCH // CHH):
        base = wid * NCH + h * CHH
        pltpu.sync_copy(ei_hbm.at[0, pl.ds(base, CHH)], sbuf)
        pltpu.sync_copy(ei_hbm.at[1, pl.ds(base, CHH)], dbuf)
        pltpu.async_copy(hs_hbm.at[sbuf.at[0]], rows0, sem0)

        def body(t, _):
            j0 = 2 * t
            pltpu.async_copy(hs_hbm.at[sbuf.at[j0 + 1]], rows1, sem1)
            pltpu.make_async_copy(hs_hbm.at[sbuf.at[j0]], rows0, sem0).wait()
            pltpu.sync_copy(rows0, aggsh.at[dbuf.at[j0]], add=True)

            @pl.when(t + 1 < CHH // 2)
            def _():
                pltpu.async_copy(hs_hbm.at[sbuf.at[j0 + 2]], rows0, sem0)

            pltpu.make_async_copy(hs_hbm.at[sbuf.at[j0 + 1]], rows1, sem1).wait()
            pltpu.sync_copy(rows1, aggsh.at[dbuf.at[j0 + 1]], add=True)
            return 0

        lax.fori_loop(0, CHH // 2, body, 0)
    plsc.subcore_barrier()
    pltpu.sync_copy(aggsh.at[pl.ds(row0, RPT)], out_hbm.at[c, pl.ds(row0, RPT)])


_scatter_kernel = functools.partial(
    pl.kernel,
    out_type=jax.ShapeDtypeStruct((NC, NP, D), jnp.float32),
    mesh=plsc.VectorSubcoreMesh(core_axis_name="c", subcore_axis_name="s"),
    scratch_types=[
        pltpu.VMEM((CHH, CH), jnp.int32),
        pltpu.VMEM((CHH, CH), jnp.int32),
        pltpu.VMEM((CH, D), jnp.float32),
        pltpu.VMEM((CH, D), jnp.float32),
        pltpu.VMEM_SHARED((NP, D), jnp.float32),
        pltpu.SemaphoreType.DMA,
        pltpu.SemaphoreType.DMA,
        pltpu.SemaphoreType.DMA,
        pltpu.SemaphoreType.DMA,
    ],
)(_scatter_body)


def _dis(degp_ref):
    deg = degp_ref[0] + degp_ref[1] + 1.0
    return lax.rsqrt(deg).reshape(BLK, 1)


def _tc1_body(x_ref, w_ref, degp_ref, hs_ref):
    h = jnp.dot(x_ref[...], w_ref[...], preferred_element_type=jnp.float32)
    hs_ref[...] = h * _dis(degp_ref)


def _tc1(x, w1, degp):
    return pl.pallas_call(
        _tc1_body,
        grid=(GRID,),
        in_specs=[
            pl.BlockSpec((BLK, D), lambda i: (i, 0)),
            pl.BlockSpec((D, D), lambda i: (0, 0)),
            pl.BlockSpec((NC, BLK), lambda i: (0, i)),
        ],
        out_specs=pl.BlockSpec((BLK, D), lambda i: (i, 0)),
        out_shape=jax.ShapeDtypeStruct((NP, D), jnp.float32),
    )(x, w1, degp)


def _tc2_body(agg_ref, hs_ref_in, degp_ref, b_ref, w_ref, hs_ref):
    dis = _dis(degp_ref)
    a = agg_ref[0] + agg_ref[1] + hs_ref_in[...]
    z = jnp.maximum(a * dis + b_ref[...], 0.0)
    h = jnp.dot(z, w_ref[...], preferred_element_type=jnp.float32)
    hs_ref[...] = h * dis


def _tc2(agg, hs, degp, b1, w2):
    return pl.pallas_call(
        _tc2_body,
        grid=(GRID,),
        in_specs=[
            pl.BlockSpec((NC, BLK, D), lambda i: (0, i, 0)),
            pl.BlockSpec((BLK, D), lambda i: (i, 0)),
            pl.BlockSpec((NC, BLK), lambda i: (0, i)),
            pl.BlockSpec((1, D), lambda i: (0, 0)),
            pl.BlockSpec((D, D), lambda i: (0, 0)),
        ],
        out_specs=pl.BlockSpec((BLK, D), lambda i: (i, 0)),
        out_shape=jax.ShapeDtypeStruct((NP, D), jnp.float32),
    )(agg, hs, degp, b1, w2)


def _tc3_body(agg_ref, hs_ref_in, degp_ref, b_ref, out_ref):
    a = agg_ref[0] + agg_ref[1] + hs_ref_in[...]
    out_ref[...] = jnp.maximum(a * _dis(degp_ref) + b_ref[...], 0.0)


def _tc3(agg, hs, degp, b2):
    return pl.pallas_call(
        _tc3_body,
        grid=(GRID,),
        in_specs=[
            pl.BlockSpec((NC, BLK, D), lambda i: (0, i, 0)),
            pl.BlockSpec((BLK, D), lambda i: (i, 0)),
            pl.BlockSpec((NC, BLK), lambda i: (0, i)),
            pl.BlockSpec((1, D), lambda i: (0, 0)),
        ],
        out_specs=pl.BlockSpec((BLK, D), lambda i: (i, 0)),
        out_shape=jax.ShapeDtypeStruct((N, D), jnp.float32),
    )(agg, hs, degp, b2)


def kernel(x, edge_index, W1, b1, W2, b2):
    ei2 = edge_index.astype(jnp.int32)
    ei = ei2.reshape(2, NCHT, CH)
    b1r = b1.reshape(1, D)
    b2r = b2.reshape(1, D)

    degp = _deg_kernel(ei)
    hs1 = _tc1(x, W1, degp)
    agg1 = _scatter_kernel(hs1, ei)
    hs2 = _tc2(agg1, hs1, degp, b1r, W2)
    agg2 = _scatter_kernel(hs2, ei)
    return _tc3(agg2, hs2, degp, b2r)


# speedup vs baseline: 1.0309x; 1.0022x over previous
import functools

import jax
import jax.numpy as jnp
from jax import lax
from jax.experimental import pallas as pl
from jax.experimental.pallas import tpu as pltpu
from jax.experimental.pallas import tpu_sc as plsc

N = 10000
NP = 10240
D = 128
E = 320000
NC = 2
NS = 16
NW = NC * NS
CH = 125
NCHT = E // CH
NCH = NCHT // NW
CHH = 40
RPT = NP // NS
BLK = 1024
GRID = NP // BLK


def _deg_body(ei_hbm, out_hbm, dbuf, buf, degsh, dsem):
    c = lax.axis_index("c")
    s = lax.axis_index("s")
    wid = c * NS + s
    row0 = s * RPT

    zero16 = jnp.zeros((16,), jnp.float32)
    one16 = jnp.ones((16,), jnp.float32)

    def fill(i, _):
        buf[pl.ds(i * 16, 16)] = zero16
        buf[pl.ds(128 + i * 16, 16)] = one16
        return 0

    lax.fori_loop(0, 128 // 16, fill, 0)
    for r in range(RPT // 128):
        pltpu.sync_copy(buf.at[pl.ds(0, 128)],
                        degsh.at[pl.ds(row0 + r * 128, 128)])
    pltpu.sync_copy(ei_hbm.at[1, pl.ds(wid * NCH, NCH)], dbuf)
    plsc.subcore_barrier()

    K = 8

    def body(g, _):
        for k in range(K):
            pltpu.async_copy(buf.at[pl.ds(128, CH)],
                             degsh.at[dbuf.at[g * K + k]], dsem, add=True)
        for k in range(K):
            pltpu.make_async_copy(buf.at[pl.ds(128, CH)],
                                  degsh.at[dbuf.at[g * K + k]], dsem).wait()
        return 0

    lax.fori_loop(0, NCH // K, body, 0)
    plsc.subcore_barrier()
    pltpu.sync_copy(degsh.at[pl.ds(row0, RPT)], out_hbm.at[c, pl.ds(row0, RPT)])


_deg_kernel = functools.partial(
    pl.kernel,
    out_type=jax.ShapeDtypeStruct((NC, NP), jnp.float32),
    mesh=plsc.VectorSubcoreMesh(core_axis_name="c", subcore_axis_name="s"),
    scratch_types=[
        pltpu.VMEM((NCH, CH), jnp.int32),
        pltpu.VMEM((256,), jnp.float32),
        pltpu.VMEM_SHARED((NP,), jnp.float32),
        pltpu.SemaphoreType.DMA,
    ],
)(_deg_body)


def _scatter_body(hs_hbm, ei_hbm, out_hbm,
                  sbuf, dbuf, rows0, rows1, aggsh, sem0, sem1, ssem0, ssem1):
    c = lax.axis_index("c")
    s = lax.axis_index("s")
    wid = c * NS + s
    row0 = s * RPT

    zero16 = jnp.zeros((16,), jnp.float32)

    def zfill(i, _):
        for k in range(D // 16):
            rows0[i, pl.ds(k * 16, 16)] = zero16
        return 0

    lax.fori_loop(0, 80, zfill, 0)
    for r in range(RPT // 80):
        pltpu.sync_copy(rows0.at[pl.ds(0, 80)],
                        aggsh.at[pl.ds(row0 + r * 80, 80)])

    plsc.subcore_barrier()

    for h in range(NCH // CHH):
        base = wid * NCH + h * CHH
        pltpu.sync_copy(ei_hbm.at[0, pl.ds(base, CHH)], sbuf)
        pltpu.sync_copy(ei_hbm.at[1, pl.ds(base, CHH)], dbuf)
        pltpu.async_copy(hs_hbm.at[sbuf.at[0]], rows0, sem0)

        def body(t, _):
            j0 = 2 * t
            pltpu.async_copy(hs_hbm.at[sbuf.at[j0 + 1]], rows1, sem1)
            pltpu.make_async_copy(hs_hbm.at[sbuf.at[j0]], rows0, sem0).wait()
            pltpu.sync_copy(rows0, aggsh.at[dbuf.at[j0]], add=True)

            @pl.when(t + 1 < CHH // 2)
            def _():
                pltpu.async_copy(hs_hbm.at[sbuf.at[j0 + 2]], rows0, sem0)

            pltpu.make_async_copy(hs_hbm.at[sbuf.at[j0 + 1]], rows1, sem1).wait()
            pltpu.sync_copy(rows1, aggsh.at[dbuf.at[j0 + 1]], add=True)
            return 0

        lax.fori_loop(0, CHH // 2, body, 0)
    plsc.subcore_barrier()
    pltpu.sync_copy(aggsh.at[pl.ds(row0, RPT)], out_hbm.at[c, pl.ds(row0, RPT)])


_scatter_kernel = functools.partial(
    pl.kernel,
    out_type=jax.ShapeDtypeStruct((NC, NP, D), jnp.float32),
    mesh=plsc.VectorSubcoreMesh(core_axis_name="c", subcore_axis_name="s"),
    scratch_types=[
        pltpu.VMEM((CHH, CH), jnp.int32),
        pltpu.VMEM((CHH, CH), jnp.int32),
        pltpu.VMEM((CH, D), jnp.float32),
        pltpu.VMEM((CH, D), jnp.float32),
        pltpu.VMEM_SHARED((NP, D), jnp.float32),
        pltpu.SemaphoreType.DMA,
        pltpu.SemaphoreType.DMA,
        pltpu.SemaphoreType.DMA,
        pltpu.SemaphoreType.DMA,
    ],
)(_scatter_body)


def _dis(degp_ref):
    deg = degp_ref[0] + degp_ref[1] + 1.0
    return lax.rsqrt(deg).reshape(BLK, 1)


def _tc1_body(x_ref, w_ref, degp_ref, hs_ref):
    h = jnp.dot(x_ref[...], w_ref[...], preferred_element_type=jnp.float32)
    hs_ref[...] = h * _dis(degp_ref)


def _tc1(x, w1, degp):
    return pl.pallas_call(
        _tc1_body,
        grid=(GRID,),
        in_specs=[
            pl.BlockSpec((BLK, D), lambda i: (i, 0)),
            pl.BlockSpec((D, D), lambda i: (0, 0)),
            pl.BlockSpec((NC, BLK), lambda i: (0, i)),
        ],
        out_specs=pl.BlockSpec((BLK, D), lambda i: (i, 0)),
        out_shape=jax.ShapeDtypeStruct((NP, D), jnp.float32),
    )(x, w1, degp)


def _tc2_body(agg_ref, hs_ref_in, degp_ref, b_ref, w_ref, hs_ref):
    dis = _dis(degp_ref)
    a = agg_ref[0] + agg_ref[1] + hs_ref_in[...]
    z = jnp.maximum(a * dis + b_ref[...], 0.0)
    h = jnp.dot(z, w_ref[...], preferred_element_type=jnp.float32)
    hs_ref[...] = h * dis


def _tc2(agg, hs, degp, b1, w2):
    return pl.pallas_call(
        _tc2_body,
        grid=(GRID,),
        in_specs=[
            pl.BlockSpec((NC, BLK, D), lambda i: (0, i, 0)),
            pl.BlockSpec((BLK, D), lambda i: (i, 0)),
            pl.BlockSpec((NC, BLK), lambda i: (0, i)),
            pl.BlockSpec((1, D), lambda i: (0, 0)),
            pl.BlockSpec((D, D), lambda i: (0, 0)),
        ],
        out_specs=pl.BlockSpec((BLK, D), lambda i: (i, 0)),
        out_shape=jax.ShapeDtypeStruct((NP, D), jnp.float32),
    )(agg, hs, degp, b1, w2)


def _tc3_body(agg_ref, hs_ref_in, degp_ref, b_ref, out_ref):
    a = agg_ref[0] + agg_ref[1] + hs_ref_in[...]
    out_ref[...] = jnp.maximum(a * _dis(degp_ref) + b_ref[...], 0.0)


def _tc3(agg, hs, degp, b2):
    return pl.pallas_call(
        _tc3_body,
        grid=(GRID,),
        in_specs=[
            pl.BlockSpec((NC, BLK, D), lambda i: (0, i, 0)),
            pl.BlockSpec((BLK, D), lambda i: (i, 0)),
            pl.BlockSpec((NC, BLK), lambda i: (0, i)),
            pl.BlockSpec((1, D), lambda i: (0, 0)),
        ],
        out_specs=pl.BlockSpec((BLK, D), lambda i: (i, 0)),
        out_shape=jax.ShapeDtypeStruct((N, D), jnp.float32),
    )(agg, hs, degp, b2)


def kernel(x, edge_index, W1, b1, W2, b2):
    ei = edge_index.astype(jnp.int32).reshape(2, NCHT, CH)
    b1r = b1.reshape(1, D)
    b2r = b2.reshape(1, D)

    degp = _deg_kernel(ei)
    hs1 = _tc1(x, W1, degp)
    agg1 = _scatter_kernel(hs1, ei)
    hs2 = _tc2(agg1, hs1, degp, b1r, W2)
    agg2 = _scatter_kernel(hs2, ei)
    return _tc3(agg2, hs2, degp, b2r)
